# baseline (speedup 1.0000x reference)
import functools

import jax
import jax.numpy as jnp
import numpy as np
from jax import lax
from jax.experimental import pallas as pl
from jax.experimental.pallas import tpu as pltpu
from jax.experimental.pallas import tpu_sc as plsc

N = 10000
E = 320000
D = 128
KEEP = 0.9

NC = 2
NS = 16
NW = NC * NS
CHUNK = 128
NCHUNK = 80
EPW = NCHUNK * CHUNK
EP = NW * EPW
NPAD = 10240
ROWS_PER_TILE = NPAD // NS

def _dropout_scale_np():
    def threefry2x32(k0, k1, x0, x1):
        x0 = x0.astype(np.uint32).copy()
        x1 = x1.astype(np.uint32).copy()
        ks0 = np.uint32(k0)
        ks1 = np.uint32(k1)
        ks2 = np.uint32(ks0 ^ ks1 ^ np.uint32(0x1BD11BDA))

        def rotl(x, d):
            return (x << np.uint32(d)) | (x >> np.uint32(32 - d))

        rot = [[13, 15, 26, 6], [17, 29, 16, 24]]
        ks = [ks0, ks1, ks2]
        x0 += ks0
        x1 += ks1
        for i in range(5):
            for d in rot[i % 2]:
                x0 += x1
                x1 = rotl(x1, d) ^ x0
            x0 += ks[(i + 1) % 3]
            x1 += ks[(i + 2) % 3] + np.uint32(i + 1)
        return x0, x1

    idx = np.arange(N * D, dtype=np.uint64)
    b1, b2 = threefry2x32(0, 42, (idx >> np.uint64(32)).astype(np.uint32),
                          idx.astype(np.uint32))
    bits = b1 ^ b2
    fbits = (bits >> np.uint32(9)) | np.uint32(0x3F800000)
    floats = fbits.view(np.float32) - np.float32(1.0)
    keep = (floats < np.float32(KEEP)).reshape(N, D)
    return np.where(keep, np.float32(1.0 / KEEP), np.float32(0.0))


_SCALE_NP = _dropout_scale_np()


def _mm_body(f_ref, s_ref, w_ref, o_ref):
    x = f_ref[...] * s_ref[...]
    o_ref[...] = jnp.dot(x, w_ref[...], preferred_element_type=jnp.float32)


def _dropout_matmul(features, scale, W):
    blk = 1000
    grid = (N // blk,)
    return pl.pallas_call(
        _mm_body,
        grid=grid,
        in_specs=[
            pl.BlockSpec((blk, D), lambda i: (i, 0)),
            pl.BlockSpec((blk, D), lambda i: (i, 0)),
            pl.BlockSpec((D, D), lambda i: (0, 0)),
        ],
        out_specs=pl.BlockSpec((blk, D), lambda i: (i, 0)),
        out_shape=jax.ShapeDtypeStruct((N, D), jnp.float32),
    )(features, scale, W)


def _sc_body(x_hbm, srcg, dag, part, src_v, rows_v, da_v, agg,
             gsem0, gsem1, ssem0, ssem1, isem0, isem1):
    cid = lax.axis_index("c")
    sid = lax.axis_index("s")
    wid = cid * NS + sid

    def _zrow(r, _):
        z = jnp.zeros((16,), jnp.float32)
        for c in range(D // 16):
            rows_v[0, r, pl.ds(c * 16, 16)] = z
        return 0

    lax.fori_loop(0, CHUNK, _zrow, 0)

    base = sid * ROWS_PER_TILE
    for k in range(ROWS_PER_TILE // CHUNK):
        pltpu.sync_copy(rows_v.at[0], agg.at[pl.ds(base + k * CHUNK, CHUNK)])

    pltpu.sync_copy(srcg.at[wid], src_v)

    plsc.subcore_barrier()

    gsem = (gsem0, gsem1)
    ssem = (ssem0, ssem1)
    isem = (isem0, isem1)

    for _h in range(2):
        pltpu.async_copy(x_hbm.at[src_v.at[0, pl.ds(_h * 64, 64)]],
                         rows_v.at[0, pl.ds(_h * 64, 64)], gsem[0])
    pltpu.async_copy(dag.at[wid, 0], da_v.at[0], isem[0])

    def _pair(jj, _):
        for b in range(2):
            j = jj * 2 + b
            rb = rows_v.at[b]
            ro = rows_v.at[1 - b]

            for h in range(2):
                pltpu.make_async_copy(
                    x_hbm.at[src_v.at[j, pl.ds(h * 64, 64)]],
                    rows_v.at[b, pl.ds(h * 64, 64)], gsem[b]).wait()
            pltpu.make_async_copy(dag.at[wid, j], da_v.at[b], isem[b]).wait()

            if b == 0:
                @pl.when(jj > 0)
                def _():
                    pltpu.make_async_copy(
                        ro, agg.at[da_v.at[1 - b, 0]], ssem[1 - b]).wait()
                for h in range(2):
                    pltpu.async_copy(
                        x_hbm.at[src_v.at[j + 1, pl.ds(h * 64, 64)]],
                        rows_v.at[1 - b, pl.ds(h * 64, 64)], gsem[1 - b])
                pltpu.async_copy(dag.at[wid, j + 1], da_v.at[1 - b], isem[1 - b])
            else:
                pltpu.make_async_copy(
                    ro, agg.at[da_v.at[1 - b, 0]], ssem[1 - b]).wait()

                @pl.when(jj < NCHUNK // 2 - 1)
                def _():
                    for h in range(2):
                        pltpu.async_copy(
                            x_hbm.at[src_v.at[j + 1, pl.ds(h * 64, 64)]],
                            rows_v.at[1 - b, pl.ds(h * 64, 64)], gsem[1 - b])
                    pltpu.async_copy(dag.at[wid, j + 1], da_v.at[1 - b],
                                     isem[1 - b])

            @plsc.parallel_loop(0, CHUNK // 16, unroll=2)
            def _scale(g):
                av = plsc.bitcast(da_v[b, 1, pl.ds(g * 16, 16)], jnp.float32)
                for l in range(16):
                    a = jnp.broadcast_to(av[l], (16,))
                    r = g * 16 + l
                    for c in range(D // 16):
                        rows_v[b, r, pl.ds(c * 16, 16)] = (
                            rows_v[b, r, pl.ds(c * 16, 16)] * a)

            pltpu.async_copy(rb, agg.at[da_v.at[b, 0]], ssem[b], add=True)
        return 0

    lax.fori_loop(0, NCHUNK // 2, _pair, 0)

    pltpu.make_async_copy(
        rows_v.at[1], agg.at[da_v.at[1, 0]], ssem[1]).wait()

    plsc.subcore_barrier()

    pltpu.sync_copy(agg.at[pl.ds(base, ROWS_PER_TILE)],
                    part.at[cid, pl.ds(base, ROWS_PER_TILE)])


def _sc_aggregate(x, srcg, dag):
    mesh = plsc.VectorSubcoreMesh(
        core_axis_name="c", subcore_axis_name="s", num_cores=NC, num_subcores=NS
    )
    return pl.kernel(
        _sc_body,
        out_type=jax.ShapeDtypeStruct((NC, NPAD, D), jnp.float32),
        mesh=mesh,
        compiler_params=pltpu.CompilerParams(needs_layout_passes=False),
        scratch_types=[
            pltpu.VMEM((NCHUNK, CHUNK), jnp.int32),
            pltpu.VMEM((2, CHUNK, D), jnp.float32),
            pltpu.VMEM((2, 2, CHUNK), jnp.int32),
            pltpu.VMEM_SHARED((NPAD, D), jnp.float32),
            pltpu.SemaphoreType.DMA,
            pltpu.SemaphoreType.DMA,
            pltpu.SemaphoreType.DMA,
            pltpu.SemaphoreType.DMA,
            pltpu.SemaphoreType.DMA,
            pltpu.SemaphoreType.DMA,
        ],
    )(x, srcg, dag)


def _combine_body(p_ref, b_ref, o_ref):
    s = p_ref[0] + p_ref[1] + b_ref[...]
    o_ref[...] = jnp.maximum(s, 0.0)


def _combine(part, b):
    blk = 1000
    grid = (N // blk,)
    return pl.pallas_call(
        _combine_body,
        grid=grid,
        in_specs=[
            pl.BlockSpec((NC, blk, D), lambda i: (0, i, 0)),
            pl.BlockSpec((1, D), lambda i: (0, 0)),
        ],
        out_specs=pl.BlockSpec((blk, D), lambda i: (i, 0)),
        out_shape=jax.ShapeDtypeStruct((N, D), jnp.float32),
    )(part, b.reshape(1, D))


def kernel(features, edge_index, adj_values, W, b):
    scale = jnp.asarray(_SCALE_NP)
    x = _dropout_matmul(features, scale, W)

    pad = EP - E
    dst = jnp.concatenate([edge_index[0], jnp.zeros((pad,), jnp.int32)])
    src = jnp.concatenate([edge_index[1], jnp.zeros((pad,), jnp.int32)])
    adj = jnp.concatenate([adj_values, jnp.zeros((pad,), jnp.float32)])
    srcg = src.reshape(NW, NCHUNK, CHUNK)
    dag = jnp.stack(
        [dst.reshape(NW, NCHUNK, CHUNK),
         lax.bitcast_convert_type(adj, jnp.int32).reshape(NW, NCHUNK, CHUNK)],
        axis=2)

    part = _sc_aggregate(x, srcg, dag)
    return _combine(part, b)


# speedup vs baseline: 1.6163x; 1.6163x over previous
import functools

import jax
import jax.numpy as jnp
import numpy as np
from jax import lax
from jax.experimental import pallas as pl
from jax.experimental.pallas import tpu as pltpu
from jax.experimental.pallas import tpu_sc as plsc

N = 10000
E = 320000
D = 128
KEEP = 0.9

NC = 2
NS = 16
NP = 2
FPT = D // (NS * NP)
NCOL = 10240
NPAD = 10240
CHUNKE = 2048
EPH = 163840
EP = NC * EPH
NCHE = EPH // CHUNKE
EPT = EPH // NS

def _dropout_scale_np():
    def threefry2x32(k0, k1, x0, x1):
        x0 = x0.astype(np.uint32).copy()
        x1 = x1.astype(np.uint32).copy()
        ks0 = np.uint32(k0)
        ks1 = np.uint32(k1)
        ks2 = np.uint32(ks0 ^ ks1 ^ np.uint32(0x1BD11BDA))

        def rotl(x, d):
            return (x << np.uint32(d)) | (x >> np.uint32(32 - d))

        rot = [[13, 15, 26, 6], [17, 29, 16, 24]]
        ks = [ks0, ks1, ks2]
        x0 += ks0
        x1 += ks1
        for i in range(5):
            for d in rot[i % 2]:
                x0 += x1
                x1 = rotl(x1, d) ^ x0
            x0 += ks[(i + 1) % 3]
            x1 += ks[(i + 2) % 3] + np.uint32(i + 1)
        return x0, x1

    idx = np.arange(N * D, dtype=np.uint64)
    b1, b2 = threefry2x32(0, 42, (idx >> np.uint64(32)).astype(np.uint32),
                          idx.astype(np.uint32))
    bits = b1 ^ b2
    fbits = (bits >> np.uint32(9)) | np.uint32(0x3F800000)
    floats = fbits.view(np.float32) - np.float32(1.0)
    keep = (floats < np.float32(KEEP)).reshape(N, D)
    return np.where(keep, np.float32(1.0 / KEEP), np.float32(0.0))


_SCALE_NP = _dropout_scale_np()


def _mm_body(f_ref, s_ref, w_ref, o_ref):
    x = f_ref[...] * s_ref[...]
    xb = jnp.dot(x, w_ref[...], preferred_element_type=jnp.float32)
    o_ref[...] = xb.T.reshape(NP, NS, FPT, xb.shape[0])


def _dropout_matmul(features, scale, W):
    blk = 1024
    grid = (NCOL // blk,)
    return pl.pallas_call(
        _mm_body,
        grid=grid,
        in_specs=[
            pl.BlockSpec((blk, D), lambda i: (i, 0)),
            pl.BlockSpec((blk, D), lambda i: (i, 0)),
            pl.BlockSpec((D, D), lambda i: (0, 0)),
        ],
        out_specs=pl.BlockSpec((NP, NS, FPT, blk), lambda i: (0, 0, 0, i)),
        out_shape=jax.ShapeDtypeStruct((NP, NS, FPT, NCOL), jnp.float32),
    )(features, scale, W)


def _sc_body(xcm, srcg, dstg, adjg, part, xt_v, agg_v, ec_src, ec_dst, ec_adj,
             se_src, se_dst, se_adj, esem0, esem1):
    cid = lax.axis_index("c")
    sid = lax.axis_index("s")

    off = sid * EPT
    pltpu.sync_copy(srcg.at[cid, pl.ds(off, EPT)], se_src.at[pl.ds(off, EPT)])
    pltpu.sync_copy(dstg.at[cid, pl.ds(off, EPT)], se_dst.at[pl.ds(off, EPT)])
    pltpu.sync_copy(adjg.at[cid, pl.ds(off, EPT)], se_adj.at[pl.ds(off, EPT)])
    plsc.subcore_barrier()

    esem = (esem0, esem1)
    cidx = [jnp.full((16,), c, jnp.int32) for c in range(FPT)]

    for p in range(NP):
        pltpu.sync_copy(xcm.at[p, sid], xt_v)

        def _zero(q, _):
            z = jnp.zeros((16,), jnp.float32)
            for c in range(FPT):
                agg_v[c, pl.ds(q * 16, 16)] = z
            return 0

        lax.fori_loop(0, NPAD // 16, _zero, 0)

        pltpu.async_copy(se_src.at[pl.ds(0, CHUNKE)], ec_src.at[0], esem[0])
        pltpu.async_copy(se_dst.at[pl.ds(0, CHUNKE)], ec_dst.at[0], esem[0])
        pltpu.async_copy(se_adj.at[pl.ds(0, CHUNKE)], ec_adj.at[0], esem[0])

        def _pair(kk, _):
            for b in range(2):
                k = kk * 2 + b
                pltpu.make_async_copy(
                    se_src.at[pl.ds(k * CHUNKE, CHUNKE)], ec_src.at[b],
                    esem[b]).wait()
                pltpu.make_async_copy(
                    se_dst.at[pl.ds(k * CHUNKE, CHUNKE)], ec_dst.at[b],
                    esem[b]).wait()
                pltpu.make_async_copy(
                    se_adj.at[pl.ds(k * CHUNKE, CHUNKE)], ec_adj.at[b],
                    esem[b]).wait()

                def _start_next():
                    nk = (k + 1) * CHUNKE
                    pltpu.async_copy(se_src.at[pl.ds(nk, CHUNKE)],
                                     ec_src.at[1 - b], esem[1 - b])
                    pltpu.async_copy(se_dst.at[pl.ds(nk, CHUNKE)],
                                     ec_dst.at[1 - b], esem[1 - b])
                    pltpu.async_copy(se_adj.at[pl.ds(nk, CHUNKE)],
                                     ec_adj.at[1 - b], esem[1 - b])

                if b == 0:
                    _start_next()
                else:
                    @pl.when(kk < NCHE // 2 - 1)
                    def _():
                        _start_next()

                @plsc.parallel_loop(0, CHUNKE // 16, unroll=2)
                def _edges(g):
                    src16 = ec_src[b, pl.ds(g * 16, 16)]
                    dst16 = ec_dst[b, pl.ds(g * 16, 16)]
                    a16 = ec_adj[b, pl.ds(g * 16, 16)]
                    for c in range(FPT):
                        v = plsc.load_gather(xt_v, [cidx[c], src16])
                        plsc.addupdate_scatter(agg_v, [cidx[c], dst16],
                                               v * a16)
            return 0

        lax.fori_loop(0, NCHE // 2, _pair, 0)

        pltpu.sync_copy(agg_v, part.at[cid, p, sid])


def _sc_aggregate(xcm, srcg, dstg, adjg):
    mesh = plsc.VectorSubcoreMesh(
        core_axis_name="c", subcore_axis_name="s", num_cores=NC, num_subcores=NS
    )
    return pl.kernel(
        _sc_body,
        out_type=jax.ShapeDtypeStruct((NC, NP, NS, FPT, NPAD), jnp.float32),
        mesh=mesh,
        compiler_params=pltpu.CompilerParams(needs_layout_passes=False),
        scratch_types=[
            pltpu.VMEM((FPT, NCOL), jnp.float32),
            pltpu.VMEM((FPT, NPAD), jnp.float32),
            pltpu.VMEM((2, CHUNKE), jnp.int32),
            pltpu.VMEM((2, CHUNKE), jnp.int32),
            pltpu.VMEM((2, CHUNKE), jnp.float32),
            pltpu.VMEM_SHARED((EPH,), jnp.int32),
            pltpu.VMEM_SHARED((EPH,), jnp.int32),
            pltpu.VMEM_SHARED((EPH,), jnp.float32),
            pltpu.SemaphoreType.DMA,
            pltpu.SemaphoreType.DMA,
        ],
    )(xcm, srcg, dstg, adjg)


def _combine_body(p_ref, b_ref, o_ref):
    s = (p_ref[0] + p_ref[1]).reshape(D, -1)
    o_ref[...] = jnp.maximum(s.T + b_ref[...], 0.0)


def _combine(part, b):
    blk = 512
    grid = (NPAD // blk,)
    return pl.pallas_call(
        _combine_body,
        grid=grid,
        in_specs=[
            pl.BlockSpec((NC, NP, NS, FPT, blk), lambda i: (0, 0, 0, 0, i)),
            pl.BlockSpec((1, D), lambda i: (0, 0)),
        ],
        out_specs=pl.BlockSpec((blk, D), lambda i: (i, 0)),
        out_shape=jax.ShapeDtypeStruct((N, D), jnp.float32),
    )(part, b.reshape(1, D))


def kernel(features, edge_index, adj_values, W, b):
    scale = jnp.asarray(_SCALE_NP)
    xcm = _dropout_matmul(features, scale, W)

    pad = EP - E
    dst = jnp.concatenate([edge_index[0], jnp.zeros((pad,), jnp.int32)])
    src = jnp.concatenate([edge_index[1], jnp.zeros((pad,), jnp.int32)])
    adj = jnp.concatenate([adj_values, jnp.zeros((pad,), jnp.float32)])
    srcg = src.reshape(NC, EPH)
    dstg = dst.reshape(NC, EPH)
    adjg = adj.reshape(NC, EPH)

    part = _sc_aggregate(xcm, srcg, dstg, adjg)
    return _combine(part, b)
